# baseline (speedup 1.0000x reference)
import functools

import jax
import jax.numpy as jnp
from jax.experimental import pallas as pl
from jax.experimental.pallas import tpu as pltpu

EPS_ = 1e-6
_ROWS = 16
_DEPTH = 3
_SPLITS = (6, 5, 5)


def _loss_kernel(p_hbm, g_hbm, b_of_row_ref, w_ref, out_ref,
                 bp_ref, bg_ref, sem_p, sem_g, acc_t_ref, acc_v_ref,
                 *, B, n_chunks):
    def start_chunk(chunk, slot):
        src_p = p_hbm.at[pl.ds(chunk * _ROWS, _ROWS)]
        src_g = g_hbm.at[pl.ds(chunk * _ROWS, _ROWS)]
        pltpu.make_async_copy(src_p, bp_ref.at[slot], sem_p.at[slot]).start(
            priority=0)
        pltpu.make_async_copy(src_g, bg_ref.at[slot], sem_g.at[slot]).start(
            priority=1)

    def wait_chunk(slot):
        pltpu.make_async_copy(
            p_hbm.at[pl.ds(0, _ROWS)], bp_ref.at[slot], sem_p.at[slot]
        ).wait()
        pltpu.make_async_copy(
            g_hbm.at[pl.ds(0, _ROWS)], bg_ref.at[slot], sem_g.at[slot]
        ).wait()

    for i in range(B):
        acc_t_ref[i] = 0.0
        acc_v_ref[i] = 0.0

    for c in range(_DEPTH - 1):
        start_chunk(c, c)

    def body(step, _):
        slot = jax.lax.rem(step, _DEPTH)

        @pl.when(step + _DEPTH - 1 < n_chunks)
        def _():
            start_chunk(step + _DEPTH - 1,
                        jax.lax.rem(step + _DEPTH - 1, _DEPTH))

        wait_chunk(slot)
        for r in range(_ROWS):
            p = bp_ref[slot, r]
            g = bg_ref[slot, r]
            pos = jnp.sum(g * p)
            s = jnp.sum(p)
            mx = jnp.max(g)
            row = step * _ROWS + r
            b = b_of_row_ref[row]
            ratio = (s - pos) / (pos + EPS_)
            is_valid = mx != 0.0
            contrib = jnp.where(is_valid, ratio * w_ref[row], 0.0)
            acc_t_ref[b] = acc_t_ref[b] + contrib
            acc_v_ref[b] = jnp.maximum(acc_v_ref[b],
                                       is_valid.astype(jnp.float32))
        return ()

    jax.lax.fori_loop(0, n_chunks, body, ())

    total = jnp.float32(0.0)
    n_valid = jnp.float32(0.0)
    for i in range(B):
        total = total + acc_t_ref[i]
        n_valid = n_valid + acc_v_ref[i]
    n = jnp.maximum(n_valid, 1.0)
    out_ref[0] = jnp.where(total == 0.0, 0.0, jnp.log(total) / n)


@jax.jit
def kernel(Y_pred, Y_gt, label):
    B, C, H, W = Y_pred.shape
    label32 = label.astype(jnp.int32)
    n_rows = B * C
    n_chunks = n_rows // _ROWS
    rows_hw = H * W // 128
    Yp = Y_pred.reshape(n_rows, rows_hw, 128)
    Yg = Y_gt.reshape(n_rows, rows_hw, 128)

    rows = jnp.arange(n_rows, dtype=jnp.int32)
    b_of_row = rows // C
    c_of_row = rows % C
    w_of_row = jnp.where(label32[b_of_row] == c_of_row,
                         jnp.float32(1.0), jnp.float32(1.0 / C))

    out = pl.pallas_call(
        functools.partial(_loss_kernel, B=B, n_chunks=n_chunks),
        in_specs=[
            pl.BlockSpec(memory_space=pl.ANY),
            pl.BlockSpec(memory_space=pl.ANY),
            pl.BlockSpec(memory_space=pltpu.SMEM),
            pl.BlockSpec(memory_space=pltpu.SMEM),
        ],
        out_specs=pl.BlockSpec(memory_space=pltpu.SMEM),
        out_shape=jax.ShapeDtypeStruct((1,), jnp.float32),
        scratch_shapes=[
            pltpu.VMEM((_DEPTH, _ROWS, rows_hw, 128), jnp.float32),
            pltpu.VMEM((_DEPTH, _ROWS, rows_hw, 128), jnp.float32),
            pltpu.SemaphoreType.DMA((_DEPTH,)),
            pltpu.SemaphoreType.DMA((_DEPTH,)),
            pltpu.SMEM((B,), jnp.float32),
            pltpu.SMEM((B,), jnp.float32),
        ],
        compiler_params=pltpu.CompilerParams(
            vmem_limit_bytes=40 * 1024 * 1024,
        ),
    )(Yp, Yg, b_of_row, w_of_row)
    return out[0]


# speedup vs baseline: 1.0692x; 1.0692x over previous
import functools

import jax
import jax.numpy as jnp
from jax.experimental import pallas as pl
from jax.experimental.pallas import tpu as pltpu

EPS_ = 1e-6
_DEPTH = 3


def _loss_kernel(p_hbm, g_hbm, w_ref, out_ref,
                 bp_ref, bg_ref, sem_p, sem_g, *, B, C):
    def start_chunk(chunk, slot):
        src_p = p_hbm.at[pl.ds(chunk * C, C)]
        src_g = g_hbm.at[pl.ds(chunk * C, C)]
        pltpu.make_async_copy(src_p, bp_ref.at[slot], sem_p.at[slot]).start(
            priority=0)
        pltpu.make_async_copy(src_g, bg_ref.at[slot], sem_g.at[slot]).start(
            priority=1)

    def wait_chunk(slot):
        pltpu.make_async_copy(
            p_hbm.at[pl.ds(0, C)], bp_ref.at[slot], sem_p.at[slot]
        ).wait()
        pltpu.make_async_copy(
            g_hbm.at[pl.ds(0, C)], bg_ref.at[slot], sem_g.at[slot]
        ).wait()

    for c in range(_DEPTH - 1):
        start_chunk(c, c)

    def body(step, carry):
        acc_t, acc_n = carry
        slot = jax.lax.rem(step, _DEPTH)

        @pl.when(step + _DEPTH - 1 < B)
        def _():
            start_chunk(step + _DEPTH - 1,
                        jax.lax.rem(step + _DEPTH - 1, _DEPTH))

        wait_chunk(slot)
        pos_rows = []
        s_rows = []
        mx_rows = []
        for r in range(C):
            p = bp_ref[slot, r]
            g = bg_ref[slot, r]
            pos_rows.append(jnp.sum(g * p, axis=0, keepdims=True))
            s_rows.append(jnp.sum(p, axis=0, keepdims=True))
            mx_rows.append(jnp.max(g, axis=0, keepdims=True))
        pos_m = jnp.concatenate(pos_rows, axis=0)
        s_m = jnp.concatenate(s_rows, axis=0)
        mx_m = jnp.concatenate(mx_rows, axis=0)
        pos_c = jnp.sum(pos_m, axis=1, keepdims=True)
        s_c = jnp.sum(s_m, axis=1, keepdims=True)
        mx_c = jnp.max(mx_m, axis=1, keepdims=True)
        ratio = (s_c - pos_c) / (pos_c + EPS_)
        w_vec = w_ref[step]
        contrib = jnp.where(mx_c != 0.0, ratio * w_vec, 0.0)
        vb = jnp.max(mx_c, axis=0, keepdims=True)
        acc_t = acc_t + contrib
        acc_n = acc_n + jnp.where(vb != 0.0, 1.0, 0.0)
        return acc_t, acc_n

    acc_t = jnp.zeros((C, 1), jnp.float32)
    acc_n = jnp.zeros((1, 1), jnp.float32)
    acc_t, acc_n = jax.lax.fori_loop(0, B, body, (acc_t, acc_n))

    total = jnp.sum(acc_t, axis=0, keepdims=True)
    n = jnp.maximum(acc_n, 1.0)
    out_ref[...] = jnp.where(total == 0.0, 0.0, jnp.log(total) / n)


@jax.jit
def kernel(Y_pred, Y_gt, label):
    B, C, H, W = Y_pred.shape
    label32 = label.astype(jnp.int32)
    n_rows = B * C
    rows_hw = H * W // 128
    Yp = Y_pred.reshape(n_rows, rows_hw, 128)
    Yg = Y_gt.reshape(n_rows, rows_hw, 128)

    cls = jnp.arange(C, dtype=jnp.int32)
    w = jnp.where(label32[:, None] == cls[None, :],
                  jnp.float32(1.0), jnp.float32(1.0 / C))
    w3 = w.reshape(B, C, 1)

    out = pl.pallas_call(
        functools.partial(_loss_kernel, B=B, C=C),
        in_specs=[
            pl.BlockSpec(memory_space=pl.ANY),
            pl.BlockSpec(memory_space=pl.ANY),
            pl.BlockSpec(memory_space=pltpu.VMEM),
        ],
        out_specs=pl.BlockSpec(memory_space=pltpu.VMEM),
        out_shape=jax.ShapeDtypeStruct((1, 1), jnp.float32),
        scratch_shapes=[
            pltpu.VMEM((_DEPTH, C, rows_hw, 128), jnp.float32),
            pltpu.VMEM((_DEPTH, C, rows_hw, 128), jnp.float32),
            pltpu.SemaphoreType.DMA((_DEPTH,)),
            pltpu.SemaphoreType.DMA((_DEPTH,)),
        ],
        compiler_params=pltpu.CompilerParams(
            vmem_limit_bytes=40 * 1024 * 1024,
        ),
    )(Yp, Yg, w3)
    return out[0, 0]
